# baseline (speedup 1.0000x reference)
import jax
import jax.numpy as jnp
from jax.experimental import pallas as pl

N = 16384
SI = 512
SJ = 2048


def _rank_kernel(mine_ref, others_ref, rank_ref):
    i = pl.program_id(0)
    j = pl.program_id(1)

    @pl.when(j == 0)
    def _init():
        rank_ref[...] = jnp.zeros_like(rank_ref)

    mine = mine_ref[...]
    others = others_ref[...]
    n_i = i * SI + jax.lax.broadcasted_iota(jnp.int32, (SI, 1), 0)
    j_idx = j * SJ + jax.lax.broadcasted_iota(jnp.int32, (SI, SJ), 1)
    lt = others < mine
    tie = (others == mine) & (j_idx < n_i)
    cnt = jnp.sum(jnp.where(lt | tie, 1, 0), axis=1, keepdims=True)
    rank_ref[...] += cnt


def _apply_kernel(rank_row_ref, col_row_ref, val_ref, idx_ref):
    k = pl.program_id(0)
    j = pl.program_id(1)

    @pl.when(j == 0)
    def _init():
        val_ref[...] = jnp.zeros_like(val_ref)
        idx_ref[...] = jnp.zeros_like(idx_ref)

    ranks = rank_row_ref[...]
    vals = col_row_ref[...]
    k_col = k * SI + jax.lax.broadcasted_iota(jnp.int32, (SI, 1), 0)
    j_idx = j * SJ + jax.lax.broadcasted_iota(jnp.int32, (SI, SJ), 1)
    hit = ranks == k_col
    val_ref[...] += jnp.sum(jnp.where(hit, vals, 0.0), axis=1, keepdims=True)
    idx_ref[...] += jnp.sum(jnp.where(hit, j_idx, 0), axis=1, keepdims=True)


def kernel(x):
    col = x[:, 0]
    col_c = col.reshape(N, 1)
    col_r = col.reshape(1, N)

    grid = (N // SI, N // SJ)
    ranks = pl.pallas_call(
        _rank_kernel,
        grid=grid,
        in_specs=[
            pl.BlockSpec((SI, 1), lambda i, j: (i, 0)),
            pl.BlockSpec((1, SJ), lambda i, j: (0, j)),
        ],
        out_specs=pl.BlockSpec((SI, 1), lambda i, j: (i, 0)),
        out_shape=jax.ShapeDtypeStruct((N, 1), jnp.int32),
    )(col_c, col_r)

    rank_row = ranks.reshape(1, N)
    vals, idx = pl.pallas_call(
        _apply_kernel,
        grid=grid,
        in_specs=[
            pl.BlockSpec((1, SJ), lambda k, j: (0, j)),
            pl.BlockSpec((1, SJ), lambda k, j: (0, j)),
        ],
        out_specs=[
            pl.BlockSpec((SI, 1), lambda k, j: (k, 0)),
            pl.BlockSpec((SI, 1), lambda k, j: (k, 0)),
        ],
        out_shape=[
            jax.ShapeDtypeStruct((N, 1), jnp.float32),
            jax.ShapeDtypeStruct((N, 1), jnp.int32),
        ],
    )(rank_row, col_r)

    return (vals.reshape(N), idx.reshape(N))


# speedup vs baseline: 56.0267x; 56.0267x over previous
import jax
import jax.numpy as jnp
from jax.experimental import pallas as pl

N = 16384
R = 128
C = 128
MASK = 0x7FFFFFFF


def _sort_kernel(col_ref, vals_ref, idx_ref):
    b = col_ref[...].view(jnp.int32)
    K = b ^ ((b >> 31) & MASK)

    rI = jax.lax.broadcasted_iota(jnp.int32, (R, C), 0)
    cI = jax.lax.broadcasted_iota(jnp.int32, (R, C), 1)
    I = rI * C + cI

    hi_m = {}
    for kd in list(range(0, 3)) + list(range(7, 14)):
        d = 1 << kd
        hi_m[kd] = ((rI & d) != 0) if kd < 3 else ((cI & (d >> 7)) != 0)
    desc_m = {}
    for ks in range(1, 15):
        s = 1 << ks
        desc_m[ks] = ((rI & s) != 0) if s < 128 else ((cI & (s >> 7)) != 0)

    for ks in range(1, 15):
        desc = desc_m[ks]
        for kd in range(ks - 1, -1, -1):
            d = 1 << kd
            if 3 <= kd < 7:
                g = R // (2 * d)
                K4 = K.reshape(g, 2, d, C)
                I4 = I.reshape(g, 2, d, C)
                d4 = desc.reshape(g, 2, d, C)[:, 0]
                loK, hiK = K4[:, 0], K4[:, 1]
                loI, hiI = I4[:, 0], I4[:, 1]
                c1 = (hiK < loK) | ((hiK == loK) & (hiI < loI))
                sw = c1 ^ d4
                K = jnp.stack([jnp.where(sw, hiK, loK),
                               jnp.where(sw, loK, hiK)], axis=1).reshape(R, C)
                I = jnp.stack([jnp.where(sw, hiI, loI),
                               jnp.where(sw, loI, hiI)], axis=1).reshape(R, C)
            else:
                hi = hi_m[kd]
                if kd < 3:
                    axis, shift = 0, d
                else:
                    axis, shift = 1, d >> 7
                pK = jnp.where(hi, jnp.roll(K, shift, axis=axis),
                               jnp.roll(K, -shift, axis=axis))
                pI = jnp.where(hi, jnp.roll(I, shift, axis=axis),
                               jnp.roll(I, -shift, axis=axis))
                c1 = (pK < K) | ((pK == K) & (pI < I))
                swap = c1 ^ hi ^ desc
                K = jnp.where(swap, pK, K)
                I = jnp.where(swap, pI, I)

    Kout = K ^ ((K >> 31) & MASK)
    vals_ref[...] = Kout.view(jnp.float32).T
    idx_ref[...] = I.T


def kernel(x):
    col = x[:, 0:1].reshape(R, C)
    vals, idx = pl.pallas_call(
        _sort_kernel,
        grid=(1,),
        in_specs=[pl.BlockSpec((R, C), lambda i: (0, 0))],
        out_specs=[
            pl.BlockSpec((R, C), lambda i: (0, 0)),
            pl.BlockSpec((R, C), lambda i: (0, 0)),
        ],
        out_shape=[
            jax.ShapeDtypeStruct((R, C), jnp.float32),
            jax.ShapeDtypeStruct((R, C), jnp.int32),
        ],
    )(col)
    return (vals.reshape(N), idx.reshape(N))


# speedup vs baseline: 56.0855x; 1.0010x over previous
import jax
import jax.numpy as jnp
from jax.experimental import pallas as pl

N = 16384
R = 128
C = 128
MASK = 0x7FFFFFFF


def _sort_kernel(col_ref, vals_ref, idx_ref):
    b = col_ref[...].view(jnp.int32)
    K = b ^ ((b >> 31) & MASK)

    rI = jax.lax.broadcasted_iota(jnp.int32, (R, C), 0)
    cI = jax.lax.broadcasted_iota(jnp.int32, (R, C), 1)
    I = rI * C + cI

    hi_m = {}
    for kd in list(range(0, 3)) + list(range(7, 14)):
        d = 1 << kd
        hi_m[kd] = ((rI & d) != 0) if kd < 3 else ((cI & (d >> 7)) != 0)
    desc_m = {}
    for ks in range(1, 15):
        s = 1 << ks
        desc_m[ks] = ((rI & s) != 0) if s < 128 else ((cI & (s >> 7)) != 0)

    for ks in range(1, 15):
        desc = desc_m[ks]
        for kd in range(ks - 1, -1, -1):
            d = 1 << kd
            if 3 <= kd < 7:
                g = R // (2 * d)
                K4 = K.reshape(g, 2, d, C)
                I4 = I.reshape(g, 2, d, C)
                d4 = desc.reshape(g, 2, d, C)[:, 0]
                loK, hiK = K4[:, 0], K4[:, 1]
                loI, hiI = I4[:, 0], I4[:, 1]
                c1 = (hiK < loK) | ((hiK == loK) & (hiI < loI))
                sw = c1 ^ d4
                K = jnp.stack([jnp.where(sw, hiK, loK),
                               jnp.where(sw, loK, hiK)], axis=1).reshape(R, C)
                I = jnp.stack([jnp.where(sw, hiI, loI),
                               jnp.where(sw, loI, hiI)], axis=1).reshape(R, C)
            else:
                hi = hi_m[kd]
                if kd < 3:
                    axis, shift = 0, d
                else:
                    axis, shift = 1, d >> 7
                pK = jnp.where(hi, jnp.roll(K, shift, axis=axis),
                               jnp.roll(K, -shift, axis=axis))
                pI = jnp.where(hi, jnp.roll(I, shift, axis=axis),
                               jnp.roll(I, -shift, axis=axis))
                c1 = (pK < K) | ((pK == K) & (pI < I))
                swap = c1 ^ hi ^ desc
                K = jnp.where(swap, pK, K)
                I = jnp.where(swap, pI, I)

    Kout = K ^ ((K >> 31) & MASK)
    vals_ref[...] = Kout.view(jnp.float32).T
    idx_ref[...] = I.T


def kernel(x):
    col = x[:, 0].reshape(R, C)
    vals, idx = pl.pallas_call(
        _sort_kernel,
        grid=(1,),
        in_specs=[pl.BlockSpec((R, C), lambda i: (0, 0))],
        out_specs=[
            pl.BlockSpec((R, C), lambda i: (0, 0)),
            pl.BlockSpec((R, C), lambda i: (0, 0)),
        ],
        out_shape=[
            jax.ShapeDtypeStruct((R, C), jnp.float32),
            jax.ShapeDtypeStruct((R, C), jnp.int32),
        ],
    )(col)
    return (vals.reshape(N), idx.reshape(N))
